# baseline (device time: 25130 ns/iter reference)
import jax
import jax.numpy as jnp
from jax import lax
from jax.experimental import pallas as pl
from jax.experimental.pallas import tpu as pltpu

N_DEV = 4
E_LOCAL = 4
N_EXPERT = 16


def kernel(x, router_W, route_idx, expert_W, shared_W):
    n, d = x.shape
    h = expert_W.shape[-1]

    def body(x_ref, rw_ref, idx_ref, ew_ref, sw_ref, out_ref,
             send_ref, comm_ref, send_sems, recv_sems):
        me = lax.axis_index("i")
        partner_a = 3 - me
        partner_b = me ^ 1

        barrier_sem = pltpu.get_barrier_semaphore()
        for nbr in (partner_a, partner_b):
            pl.semaphore_signal(
                barrier_sem, inc=1,
                device_id=(nbr,), device_id_type=pl.DeviceIdType.MESH,
            )
        pl.semaphore_wait(barrier_sem, 2)

        xf = x_ref[:, :]
        scores = jnp.dot(xf, rw_ref[:, :], preferred_element_type=jnp.float32)
        s_max = jnp.max(scores, axis=-1, keepdims=True)
        p = jnp.exp(scores - s_max)
        probs = p / jnp.sum(p, axis=-1, keepdims=True)
        idx = idx_ref[:, :]
        onehot = idx == lax.broadcasted_iota(jnp.int32, (n, N_EXPERT), 1)
        chosen_p = jnp.sum(jnp.where(onehot, probs, 0.0), axis=1,
                           keepdims=True)

        xb = xf.astype(jnp.bfloat16)
        acc = jnp.zeros((n, h), jnp.float32)
        for le in range(E_LOCAL):
            e_global = me * E_LOCAL + le
            gate = jnp.where(idx[:, :1] == e_global, chosen_p, 0.0)
            y = jnp.dot(xb, ew_ref[le, :, :].astype(jnp.bfloat16),
                        preferred_element_type=jnp.float32)
            acc = acc + gate * y

        shared = jnp.dot(xb, sw_ref[:, :].astype(jnp.bfloat16),
                         preferred_element_type=jnp.float32)

        send_ref[0, :, :] = acc.astype(jnp.bfloat16)
        rdma_a = pltpu.make_async_remote_copy(
            src_ref=send_ref.at[0],
            dst_ref=comm_ref.at[0],
            send_sem=send_sems.at[0],
            recv_sem=recv_sems.at[0],
            device_id=(partner_a,),
            device_id_type=pl.DeviceIdType.MESH,
        )
        rdma_a.start()
        rdma_a.wait()
        acc = acc + comm_ref[0, :, :].astype(jnp.float32)

        send_ref[1, :, :] = acc.astype(jnp.bfloat16)
        rdma_b = pltpu.make_async_remote_copy(
            src_ref=send_ref.at[1],
            dst_ref=comm_ref.at[1],
            send_sem=send_sems.at[1],
            recv_sem=recv_sems.at[1],
            device_id=(partner_b,),
            device_id_type=pl.DeviceIdType.MESH,
        )
        rdma_b.start()
        rdma_b.wait()
        acc = acc + comm_ref[1, :, :].astype(jnp.float32)

        out_ref[:, :] = acc + shared

    return pl.pallas_call(
        body,
        out_shape=jax.ShapeDtypeStruct((n, h), jnp.float32),
        in_specs=[pl.BlockSpec(memory_space=pltpu.VMEM)] * 5,
        out_specs=pl.BlockSpec(memory_space=pltpu.VMEM),
        scratch_shapes=[
            pltpu.VMEM((2, n, h), jnp.bfloat16),
            pltpu.VMEM((2, n, h), jnp.bfloat16),
            pltpu.SemaphoreType.DMA((2,)),
            pltpu.SemaphoreType.DMA((2,)),
        ],
        compiler_params=pltpu.CompilerParams(collective_id=0),
    )(x, router_W, route_idx, expert_W, shared_W)


# device time: 19698 ns/iter; 1.2758x vs baseline; 1.2758x over previous
import jax
import jax.numpy as jnp
from jax import lax
from jax.experimental import pallas as pl
from jax.experimental.pallas import tpu as pltpu

N_DEV = 4
E_LOCAL = 4
N_EXPERT = 16


def kernel(x, router_W, route_idx, expert_W, shared_W):
    n, d = x.shape
    h = expert_W.shape[-1]

    def body(x_ref, rw_ref, idx_ref, ew_ref, sw_ref, out_ref,
             send_ref, comm_ref, send_sems, recv_sems):
        me = lax.axis_index("i")
        partner_a = 3 - me
        partner_b = me ^ 1

        barrier_sem = pltpu.get_barrier_semaphore()
        for nbr in (partner_a, partner_b):
            pl.semaphore_signal(
                barrier_sem, inc=1,
                device_id=(nbr,), device_id_type=pl.DeviceIdType.MESH,
            )
        pl.semaphore_wait(barrier_sem, 2)

        xf = x_ref[:, :]
        scores = jnp.dot(xf, rw_ref[:, :], preferred_element_type=jnp.float32)
        s_max = jnp.max(scores, axis=-1, keepdims=True)
        p = jnp.exp(scores - s_max)
        probs = p / jnp.sum(p, axis=-1, keepdims=True)
        idx = idx_ref[:, :]
        onehot = idx == lax.broadcasted_iota(jnp.int32, (n, N_EXPERT), 1)
        chosen_p = jnp.sum(jnp.where(onehot, probs, 0.0), axis=1,
                           keepdims=True)

        hh = h // 2
        xb = xf.astype(jnp.bfloat16)
        gates = []
        for le in range(E_LOCAL):
            e_global = me * E_LOCAL + le
            gates.append(jnp.where(idx[:, :1] == e_global, chosen_p, 0.0))

        def partial_half(lo):
            a = jnp.zeros((n, hh), jnp.float32)
            for le in range(E_LOCAL):
                y = jnp.dot(xb, ew_ref[le, :, lo:lo + hh].astype(jnp.bfloat16),
                            preferred_element_type=jnp.float32)
                a = a + gates[le] * y
            return a

        def exchange(slot, src, target):
            return pltpu.make_async_remote_copy(
                src_ref=src,
                dst_ref=comm_ref.at[slot],
                send_sem=send_sems.at[slot],
                recv_sem=recv_sems.at[slot],
                device_id=(target,),
                device_id_type=pl.DeviceIdType.MESH,
            )

        acc0 = partial_half(0)
        send_ref[0, :, :] = acc0.astype(jnp.bfloat16)
        rdma0 = exchange(0, send_ref.at[0], partner_a)
        rdma0.start()

        acc1 = partial_half(hh)
        send_ref[1, :, :] = acc1.astype(jnp.bfloat16)
        rdma1 = exchange(1, send_ref.at[1], partner_b)
        rdma1.start()

        shared = jnp.dot(xb, sw_ref[:, :].astype(jnp.bfloat16),
                         preferred_element_type=jnp.float32)

        rdma0.wait()
        acc0 = acc0 + comm_ref[0, :, :].astype(jnp.float32)
        send_ref[2, :, :] = acc0.astype(jnp.bfloat16)
        rdma2 = exchange(2, send_ref.at[2], partner_b)
        rdma2.start()

        rdma1.wait()
        acc1 = acc1 + comm_ref[1, :, :].astype(jnp.float32)
        send_ref[3, :, :] = acc1.astype(jnp.bfloat16)
        rdma3 = exchange(3, send_ref.at[3], partner_a)
        rdma3.start()

        rdma2.wait()
        out_ref[:, :hh] = acc0 + comm_ref[2, :, :].astype(jnp.float32) \
            + shared[:, :hh]
        rdma3.wait()
        out_ref[:, hh:] = acc1 + comm_ref[3, :, :].astype(jnp.float32) \
            + shared[:, hh:]

    return pl.pallas_call(
        body,
        out_shape=jax.ShapeDtypeStruct((n, h), jnp.float32),
        in_specs=[pl.BlockSpec(memory_space=pltpu.VMEM)] * 5,
        out_specs=pl.BlockSpec(memory_space=pltpu.VMEM),
        scratch_shapes=[
            pltpu.VMEM((4, n, h // 2), jnp.bfloat16),
            pltpu.VMEM((4, n, h // 2), jnp.bfloat16),
            pltpu.SemaphoreType.DMA((4,)),
            pltpu.SemaphoreType.DMA((4,)),
        ],
        compiler_params=pltpu.CompilerParams(collective_id=0),
    )(x, router_W, route_idx, expert_W, shared_W)


# device time: 18488 ns/iter; 1.3593x vs baseline; 1.0654x over previous
import jax
import jax.numpy as jnp
from jax import lax
from jax.experimental import pallas as pl
from jax.experimental.pallas import tpu as pltpu

N_DEV = 4
E_LOCAL = 4
N_EXPERT = 16


def kernel(x, router_W, route_idx, expert_W, shared_W):
    n, d = x.shape
    h = expert_W.shape[-1]

    def body(x_ref, rw_ref, idx_ref, ew_ref, sw_ref, out_ref,
             send_ref, comm_ref, send_sems, recv_sems):
        me = lax.axis_index("i")
        partner_a = 3 - me
        partner_b = me ^ 1

        barrier_sem = pltpu.get_barrier_semaphore()
        for nbr in (partner_a, partner_b):
            pl.semaphore_signal(
                barrier_sem, inc=1,
                device_id=(nbr,), device_id_type=pl.DeviceIdType.MESH,
            )
        pl.semaphore_wait(barrier_sem, 2)

        xf = x_ref[:, :]
        scores = jnp.dot(xf, rw_ref[:, :], preferred_element_type=jnp.float32)
        s_max = jnp.max(scores, axis=-1, keepdims=True)
        p = jnp.exp(scores - s_max)
        probs = p / jnp.sum(p, axis=-1, keepdims=True)
        idx = idx_ref[:, :]
        onehot = idx == lax.broadcasted_iota(jnp.int32, (n, N_EXPERT), 1)
        chosen_p = jnp.sum(jnp.where(onehot, probs, 0.0), axis=1,
                           keepdims=True)

        NQ = 4
        qw = h // NQ
        xb = xf.astype(jnp.bfloat16)
        gates = []
        for le in range(E_LOCAL):
            e_global = me * E_LOCAL + le
            gates.append(jnp.where(idx[:, :1] == e_global, chosen_p, 0.0))

        def partial_quarter(q):
            lo = q * qw
            a = jnp.zeros((n, qw), jnp.float32)
            for le in range(E_LOCAL):
                y = jnp.dot(xb, ew_ref[le, :, lo:lo + qw].astype(jnp.bfloat16),
                            preferred_element_type=jnp.float32)
                a = a + gates[le] * y
            return a

        def exchange(slot, target):
            return pltpu.make_async_remote_copy(
                src_ref=send_ref.at[slot],
                dst_ref=comm_ref.at[slot],
                send_sem=send_sems.at[slot],
                recv_sem=recv_sems.at[slot],
                device_id=(target,),
                device_id_type=pl.DeviceIdType.MESH,
            )

        first = [partner_a, partner_b, partner_a, partner_b]
        second = [partner_b, partner_a, partner_b, partner_a]

        accs = [None] * NQ
        p1 = [None] * NQ
        for q in range(NQ):
            accs[q] = partial_quarter(q)
            send_ref[q, :, :] = accs[q].astype(jnp.bfloat16)
            p1[q] = exchange(q, first[q])
            p1[q].start()

        shared = jnp.dot(xb, sw_ref[:, :].astype(jnp.bfloat16),
                         preferred_element_type=jnp.float32)

        p2 = [None] * NQ
        for q in range(NQ):
            p1[q].wait()
            accs[q] = accs[q] + comm_ref[q, :, :].astype(jnp.float32)
            send_ref[NQ + q, :, :] = accs[q].astype(jnp.bfloat16)
            p2[q] = exchange(NQ + q, second[q])
            p2[q].start()

        for q in range(NQ):
            p2[q].wait()
            lo = q * qw
            out_ref[:, lo:lo + qw] = (
                accs[q] + comm_ref[NQ + q, :, :].astype(jnp.float32)
                + shared[:, lo:lo + qw]
            )

    return pl.pallas_call(
        body,
        out_shape=jax.ShapeDtypeStruct((n, h), jnp.float32),
        in_specs=[pl.BlockSpec(memory_space=pltpu.VMEM)] * 5,
        out_specs=pl.BlockSpec(memory_space=pltpu.VMEM),
        scratch_shapes=[
            pltpu.VMEM((8, n, h // 4), jnp.bfloat16),
            pltpu.VMEM((8, n, h // 4), jnp.bfloat16),
            pltpu.SemaphoreType.DMA((8,)),
            pltpu.SemaphoreType.DMA((8,)),
        ],
        compiler_params=pltpu.CompilerParams(collective_id=0),
    )(x, router_W, route_idx, expert_W, shared_W)


# device time: 17975 ns/iter; 1.3981x vs baseline; 1.0285x over previous
import jax
import jax.numpy as jnp
from jax import lax
from jax.experimental import pallas as pl
from jax.experimental.pallas import tpu as pltpu

N_DEV = 4
E_LOCAL = 4
N_EXPERT = 16
NQ = 4


def kernel(x, router_W, route_idx, expert_W, shared_W):
    n, d = x.shape
    h = expert_W.shape[-1]

    def body(x_ref, rw_ref, idx_ref, ew_ref, sw_ref, out_ref,
             send_ref, comm_ref, send_sems, recv_sems):
        me = lax.axis_index("i")
        partner_a = 3 - me
        partner_b = me ^ 1

        barrier_sem = pltpu.get_barrier_semaphore()
        for nbr in (partner_a, partner_b):
            pl.semaphore_signal(
                barrier_sem, inc=1,
                device_id=(nbr,), device_id_type=pl.DeviceIdType.MESH,
            )

        xf = x_ref[:, :]
        scores = jnp.dot(xf, rw_ref[:, :], preferred_element_type=jnp.float32)
        s_max = jnp.max(scores, axis=-1, keepdims=True)
        p = jnp.exp(scores - s_max)
        probs = p / jnp.sum(p, axis=-1, keepdims=True)
        idx = idx_ref[:, :]
        onehot = idx == lax.broadcasted_iota(jnp.int32, (n, N_EXPERT), 1)
        chosen_p = jnp.sum(jnp.where(onehot, probs, 0.0), axis=1,
                           keepdims=True)

        qw = h // NQ
        xb = xf.astype(jnp.bfloat16)
        gates = []
        for le in range(E_LOCAL):
            e_global = me * E_LOCAL + le
            gates.append(jnp.where(idx[:, :1] == e_global, chosen_p, 0.0))

        def partial_quarter(q):
            lo = q * qw
            a = jnp.zeros((n, qw), jnp.float32)
            for le in range(E_LOCAL):
                y = jnp.dot(xb, ew_ref[le, :, lo:lo + qw].astype(jnp.bfloat16),
                            preferred_element_type=jnp.float32)
                a = a + gates[le] * y
            return a

        def exchange(slot, target):
            return pltpu.make_async_remote_copy(
                src_ref=send_ref.at[slot],
                dst_ref=comm_ref.at[slot],
                send_sem=send_sems.at[slot],
                recv_sem=recv_sems.at[slot],
                device_id=(target,),
                device_id_type=pl.DeviceIdType.MESH,
            )

        first = [partner_a if q % 2 == 0 else partner_b for q in range(NQ)]
        second = [partner_b if q % 2 == 0 else partner_a for q in range(NQ)]

        accs = [None] * NQ
        p1 = [None] * NQ
        for q in range(NQ):
            accs[q] = partial_quarter(q)
            send_ref[q, :, :] = accs[q].astype(jnp.bfloat16)
            if q == 0:
                pl.semaphore_wait(barrier_sem, 2)
            p1[q] = exchange(q, first[q])
            p1[q].start()

        shared = jnp.dot(xb, sw_ref[:, :].astype(jnp.bfloat16),
                         preferred_element_type=jnp.float32)

        p2 = [None] * NQ
        for q in range(NQ):
            p1[q].wait_recv()
            accs[q] = accs[q] + comm_ref[q, :, :].astype(jnp.float32)
            send_ref[NQ + q, :, :] = accs[q].astype(jnp.bfloat16)
            p2[q] = exchange(NQ + q, second[q])
            p2[q].start()

        for q in range(NQ):
            p2[q].wait_recv()
            lo = q * qw
            out_ref[:, lo:lo + qw] = (
                accs[q] + comm_ref[NQ + q, :, :].astype(jnp.float32)
                + shared[:, lo:lo + qw]
            )

        for q in range(NQ):
            p1[q].wait_send()
            p2[q].wait_send()

    return pl.pallas_call(
        body,
        out_shape=jax.ShapeDtypeStruct((n, h), jnp.float32),
        in_specs=[pl.BlockSpec(memory_space=pltpu.VMEM)] * 5,
        out_specs=pl.BlockSpec(memory_space=pltpu.VMEM),
        scratch_shapes=[
            pltpu.VMEM((2 * NQ, n, h // NQ), jnp.bfloat16),
            pltpu.VMEM((2 * NQ, n, h // NQ), jnp.bfloat16),
            pltpu.SemaphoreType.DMA((2 * NQ,)),
            pltpu.SemaphoreType.DMA((2 * NQ,)),
        ],
        compiler_params=pltpu.CompilerParams(collective_id=0),
    )(x, router_W, route_idx, expert_W, shared_W)
